# baseline (speedup 1.0000x reference)
import jax
import jax.numpy as jnp
from jax.experimental import pallas as pl
from jax.experimental.pallas import tpu as pltpu

_B, _N, _C, _H, _W = 2, 512, 256, 7, 7
_D_IN = _C * _H * _W
_D_HID = 1024
_OUT = 16

_TM = 1024
_TK = 1792


def _mlp_kernel(x_ref, w1_ref, b1_ref, w2_ref, b2_ref, wh_ref, bh_ref,
                o_ref, acc_ref):
    k = pl.program_id(1)

    @pl.when(k == 0)
    def _init():
        acc_ref[...] = jnp.zeros_like(acc_ref)

    acc_ref[...] += jnp.dot(x_ref[...], w1_ref[...],
                            preferred_element_type=jnp.float32)

    @pl.when(k == pl.num_programs(1) - 1)
    def _finish():
        h1 = jnp.maximum(acc_ref[...] + b1_ref[...], 0.0)
        h2 = jnp.maximum(
            jnp.dot(h1, w2_ref[...], preferred_element_type=jnp.float32)
            + b2_ref[...], 0.0)
        o_ref[...] = (jnp.dot(h2, wh_ref[...],
                              preferred_element_type=jnp.float32)
                      + bh_ref[...])


def kernel(aligned_feat, W1, b1, W2, b2, Wc, bc, Wr, br):
    M = _B * _N
    x = aligned_feat.reshape(M, _D_IN)
    Wh = jnp.concatenate([Wc, Wr], axis=1)
    bh = jnp.concatenate([bc, br]).reshape(1, _OUT)
    b1r = b1.reshape(1, _D_HID)
    b2r = b2.reshape(1, _D_HID)

    grid = (M // _TM, _D_IN // _TK)
    out = pl.pallas_call(
        _mlp_kernel,
        grid=grid,
        in_specs=[
            pl.BlockSpec((_TM, _TK), lambda m, k: (m, k)),
            pl.BlockSpec((_TK, _D_HID), lambda m, k: (k, 0)),
            pl.BlockSpec((1, _D_HID), lambda m, k: (0, 0)),
            pl.BlockSpec((_D_HID, _D_HID), lambda m, k: (0, 0)),
            pl.BlockSpec((1, _D_HID), lambda m, k: (0, 0)),
            pl.BlockSpec((_D_HID, _OUT), lambda m, k: (0, 0)),
            pl.BlockSpec((1, _OUT), lambda m, k: (0, 0)),
        ],
        out_specs=pl.BlockSpec((_TM, _OUT), lambda m, k: (m, 0)),
        out_shape=jax.ShapeDtypeStruct((M, _OUT), jnp.float32),
        scratch_shapes=[pltpu.VMEM((_TM, _D_HID), jnp.float32)],
        compiler_params=pltpu.CompilerParams(
            dimension_semantics=("parallel", "arbitrary")),
    )(x, W1, b1r, W2, b2r, Wh, bh)
    return out.reshape(_B, _N, _OUT)
